# baseline (speedup 1.0000x reference)
import functools

import jax
import jax.numpy as jnp
from jax.experimental import pallas as pl
from jax.experimental.pallas import tpu as pltpu
from jax.experimental.pallas import tpu_sc as plsc

EMBED_DIM = 16
NIDX = 16
ROWS_PER_TILE = 8


@functools.partial(
    pl.kernel,
    out_type=jax.ShapeDtypeStruct((EMBED_DIM,), jnp.float32),
    mesh=plsc.VectorSubcoreMesh(
        core_axis_name="c", subcore_axis_name="s", num_cores=1, num_subcores=1
    ),
    scratch_types=[
        pltpu.VMEM((NIDX,), jnp.int32),
        pltpu.VMEM((ROWS_PER_TILE, EMBED_DIM), jnp.float32),
        pltpu.VMEM((EMBED_DIM,), jnp.float32),
    ],
    compiler_params=pltpu.CompilerParams(use_tc_tiling_on_sc=False),
)
def _row_gather(table_hbm, ids_hbm, out_hbm, idx_v, tile_v, row_v):
    pltpu.sync_copy(ids_hbm.at[0, pl.ds(0, NIDX)], idx_v)
    idx = idx_v[...][0]
    base = (idx // ROWS_PER_TILE) * ROWS_PER_TILE
    pltpu.sync_copy(table_hbm.at[pl.ds(base, ROWS_PER_TILE)], tile_v)
    sub = idx - base
    row = tile_v[0, :]
    for j in range(1, ROWS_PER_TILE):
        row = jnp.where(sub == j, tile_v[j, :], row)
    row_v[...] = row
    pltpu.sync_copy(row_v, out_hbm)


def kernel(ids, table):
    ids_head = jax.lax.slice(ids, (0, 0), (1, NIDX)).astype(jnp.int32)
    return _row_gather(table, ids_head)


# speedup vs baseline: 26.1291x; 26.1291x over previous
import functools

import jax
import jax.numpy as jnp
from jax.experimental import pallas as pl
from jax.experimental.pallas import tpu as pltpu
from jax.experimental.pallas import tpu_sc as plsc

EMBED_DIM = 16
NIDX = 16
LANES = 128
HALF = 8


@functools.partial(
    pl.kernel,
    out_type=jax.ShapeDtypeStruct((EMBED_DIM,), jnp.float32),
    mesh=plsc.ScalarSubcoreMesh(axis_name="c", num_cores=1),
    scratch_types=[
        pltpu.SMEM((NIDX,), jnp.int32),
        pltpu.SMEM((HALF, LANES), jnp.float32),
        pltpu.SMEM((HALF, LANES), jnp.float32),
        pltpu.SemaphoreType.DMA,
        pltpu.SemaphoreType.DMA,
        pltpu.SMEM((EMBED_DIM,), jnp.float32),
    ],
)
def _row_lookup(
    table_t_hbm, ids_t_hbm, out_hbm, idx_s, win0_s, win1_s, sem0, sem1, out_s
):
    pltpu.sync_copy(ids_t_hbm.at[0, pl.ds(0, NIDX)], idx_s)
    idx = idx_s[0]
    base = pl.multiple_of((idx // LANES) * LANES, LANES)
    col = idx - base
    c0 = pltpu.async_copy(
        table_t_hbm.at[pl.ds(0, HALF), pl.ds(base, LANES)], win0_s, sem0
    )
    c1 = pltpu.async_copy(
        table_t_hbm.at[pl.ds(HALF, HALF), pl.ds(base, LANES)], win1_s, sem1
    )
    c0.wait()
    for r in range(HALF):
        out_s[r] = win0_s[r, col]
    c1.wait()
    for r in range(HALF):
        out_s[HALF + r] = win1_s[r, col]
    pltpu.sync_copy(out_s, out_hbm)


def kernel(ids, table):
    return _row_lookup(table.T, ids.T.astype(jnp.int32))
